# baseline (speedup 1.0000x reference)
import functools

import jax
import jax.numpy as jnp
import numpy as np
from jax import lax
from jax.experimental import pallas as pl
from jax.experimental.pallas import tpu as pltpu
from jax.experimental.pallas import tpu_sc as plsc

_N = 4096
_TOP_K = 10
_L = 40
_R = 256

_DISC = np.where(
    np.arange(_L) < _TOP_K,
    1.0 / np.log2(2.0 + np.arange(_L, dtype=np.float64)),
    0.0,
).astype(np.float32)


def _extract_kernel(x_ref, y_ref, lam_ref, fidx_ref):
    i = pl.program_id(0)
    x = x_ref[...]
    y = y_ref[...]
    lane = jax.lax.broadcasted_iota(jnp.int32, (_R, _N), 1)
    grow = jax.lax.broadcasted_iota(jnp.int32, (_R, _N), 0) + i * _R
    diag = lane == grow
    neg = jnp.float32(-1.0)
    ye = jnp.where(diag, neg, y)
    xe = jnp.where(diag, neg, x)

    ys_cols, yi_cols, xc_cols = [], [], []
    xv_cols, xprev_cols, xcum_cols = [], [], []
    xcum = jnp.zeros((_R, 1), jnp.int32)
    for _ in range(_L):
        ym = jnp.max(ye, axis=1, keepdims=True)
        yi = jnp.min(jnp.where(ye == ym, lane, _N), axis=1, keepdims=True)
        oh = lane == yi
        xc = jnp.sum(jnp.where(oh, x, 0.0), axis=1, keepdims=True)
        ye = jnp.where(oh, neg, ye)
        ys_cols.append(ym)
        yi_cols.append(yi)
        xc_cols.append(xc)
        xm = jnp.max(xe, axis=1, keepdims=True)
        eqx = xe == xm
        cnt = jnp.sum(eqx.astype(jnp.int32), axis=1, keepdims=True)
        xe = jnp.where(eqx, neg, xe)
        xv_cols.append(xm)
        xprev_cols.append(xcum)
        xcum = xcum + cnt
        xcum_cols.append(xcum)

    ys = jnp.concatenate(ys_cols, axis=1)
    yi = jnp.concatenate(yi_cols, axis=1)
    xc = jnp.concatenate(xc_cols, axis=1)

    pos40 = jax.lax.broadcasted_iota(jnp.int32, (_R, _L), 1)
    xs = jnp.zeros((_R, _L), jnp.float32)
    for t in range(_L):
        m = (xprev_cols[t] <= pos40) & (pos40 < xcum_cols[t])
        xs = jnp.where(m, xv_cols[t], xs)

    pos = jax.lax.broadcasted_iota(jnp.int32, (1, _L), 1).astype(jnp.float32)
    disc = jnp.where(pos < _TOP_K, 1.0 / jnp.log2(2.0 + pos), 0.0)
    idcg = jnp.sum(
        (jnp.exp2(xs[:, :_TOP_K]) - 1.0) * disc[:, :_TOP_K],
        axis=1,
        keepdims=True,
    )
    gain = jnp.exp2(xc) - 1.0

    lam = jnp.zeros((_R, _L), jnp.float32)
    for k in range(_L):
        sx = jnp.sign(xs - xs[:, k : k + 1])
        f1 = -1.0 / (1.0 + jnp.exp(ys - ys[:, k : k + 1]))
        nd = jnp.abs((gain - gain[:, k : k + 1]) * (disc - float(_DISC[k])))
        lam = lam + sx * f1 * nd
    lam_ref[...] = lam / idcg

    grow40 = jax.lax.broadcasted_iota(jnp.int32, (_R, _L), 0) + i * _R
    fidx_ref[...] = yi + grow40 * _N


def _tc_extract(x_similarity, y_similarity):
    grid = (_N // _R,)
    return pl.pallas_call(
        _extract_kernel,
        grid=grid,
        in_specs=[
            pl.BlockSpec((_R, _N), lambda i: (i, 0)),
            pl.BlockSpec((_R, _N), lambda i: (i, 0)),
        ],
        out_specs=[
            pl.BlockSpec((_R, _L), lambda i: (i, 0)),
            pl.BlockSpec((_R, _L), lambda i: (i, 0)),
        ],
        out_shape=[
            jax.ShapeDtypeStruct((_N, _L), jnp.float32),
            jax.ShapeDtypeStruct((_N, _L), jnp.int32),
        ],
        compiler_params=pltpu.CompilerParams(
            dimension_semantics=("parallel",)
        ),
    )(x_similarity, y_similarity)


_NW = 32
_ROWS_PER_W = _N // _NW
_VALS_PER_W = _ROWS_PER_W * _L
_SCAT_CHUNK = 128
_ZCHUNK = 16384
_SLAB = _ROWS_PER_W * _N


def _sc_scatter(fidx_flat, lam_flat, zchunk):
    mesh = plsc.VectorSubcoreMesh(core_axis_name="c", subcore_axis_name="s")

    @functools.partial(
        pl.kernel,
        out_type=jax.ShapeDtypeStruct((_N * _N,), jnp.float32),
        mesh=mesh,
        scratch_types=[
            pltpu.VMEM((_VALS_PER_W,), jnp.int32),
            pltpu.VMEM((_VALS_PER_W,), jnp.float32),
            pltpu.SemaphoreType.DMA,
            pltpu.SemaphoreType.DMA,
        ],
    )
    def scatter_kernel(fidx_hbm, lam_hbm, z_hbm, out_hbm, idx_v, val_v, zsem, ssem):
        wid = lax.axis_index("s") * 2 + lax.axis_index("c")
        slab = wid * _SLAB
        zcopies = [
            pltpu.async_copy(
                z_hbm, out_hbm.at[pl.ds(slab + k * _ZCHUNK, _ZCHUNK)], zsem
            )
            for k in range(_SLAB // _ZCHUNK)
        ]
        base = wid * _VALS_PER_W
        pltpu.sync_copy(fidx_hbm.at[pl.ds(base, _VALS_PER_W)], idx_v)
        pltpu.sync_copy(lam_hbm.at[pl.ds(base, _VALS_PER_W)], val_v)
        for c in zcopies:
            c.wait()
        scopies = [
            pltpu.async_copy(
                val_v.at[pl.ds(c * _SCAT_CHUNK, _SCAT_CHUNK)],
                out_hbm.at[idx_v.at[pl.ds(c * _SCAT_CHUNK, _SCAT_CHUNK)]],
                ssem,
            )
            for c in range(_VALS_PER_W // _SCAT_CHUNK)
        ]
        for c in scopies:
            c.wait()

    return scatter_kernel(fidx_flat, lam_flat, zchunk)


@jax.jit
def kernel(x_similarity, y_similarity):
    lam, fidx = _tc_extract(x_similarity, y_similarity)
    zchunk = jnp.zeros((_ZCHUNK,), jnp.float32)
    out_flat = _sc_scatter(fidx.reshape(-1), lam.reshape(-1), zchunk)
    return out_flat.reshape(_N, _N)


# speedup vs baseline: 1.9360x; 1.9360x over previous
import functools

import jax
import jax.numpy as jnp
import numpy as np
from jax import lax
from jax.experimental import pallas as pl
from jax.experimental.pallas import tpu as pltpu
from jax.experimental.pallas import tpu_sc as plsc

_N = 4096
_TOP_K = 10
_L = 40
_R = 256

_DISC = np.where(
    np.arange(_L) < _TOP_K,
    1.0 / np.log2(2.0 + np.arange(_L, dtype=np.float64)),
    0.0,
).astype(np.float32)


def _extract_kernel(x_ref, y_ref, lam_ref, fidx_ref):
    i = pl.program_id(0)
    x = x_ref[...]
    y = y_ref[...]
    lane = jax.lax.broadcasted_iota(jnp.int32, (_R, _N), 1)
    grow = jax.lax.broadcasted_iota(jnp.int32, (_R, _N), 0) + i * _R
    diag = lane == grow
    neg = jnp.float32(-1.0)
    ye = jnp.where(diag, neg, y)
    xe = jnp.where(diag, neg, x)

    ys_cols, yi_cols, xc_cols = [], [], []
    xv_cols, xprev_cols, xcum_cols = [], [], []
    xcum = jnp.zeros((_R, 1), jnp.int32)
    for _ in range(_L):
        ym = jnp.max(ye, axis=1, keepdims=True)
        yi = jnp.min(jnp.where(ye == ym, lane, _N), axis=1, keepdims=True)
        oh = lane == yi
        xc = jnp.sum(jnp.where(oh, x, 0.0), axis=1, keepdims=True)
        ye = jnp.where(oh, neg, ye)
        ys_cols.append(ym)
        yi_cols.append(yi)
        xc_cols.append(xc)
        xm = jnp.max(xe, axis=1, keepdims=True)
        eqx = xe == xm
        cnt = jnp.sum(eqx.astype(jnp.int32), axis=1, keepdims=True)
        xe = jnp.where(eqx, neg, xe)
        xv_cols.append(xm)
        xprev_cols.append(xcum)
        xcum = xcum + cnt
        xcum_cols.append(xcum)

    ys = jnp.concatenate(ys_cols, axis=1)
    yi = jnp.concatenate(yi_cols, axis=1)
    xc = jnp.concatenate(xc_cols, axis=1)

    pos40 = jax.lax.broadcasted_iota(jnp.int32, (_R, _L), 1)
    xs = jnp.zeros((_R, _L), jnp.float32)
    for t in range(_L):
        m = (xprev_cols[t] <= pos40) & (pos40 < xcum_cols[t])
        xs = jnp.where(m, xv_cols[t], xs)

    pos = jax.lax.broadcasted_iota(jnp.int32, (1, _L), 1).astype(jnp.float32)
    disc = jnp.where(pos < _TOP_K, 1.0 / jnp.log2(2.0 + pos), 0.0)
    idcg = jnp.sum(
        (jnp.exp2(xs[:, :_TOP_K]) - 1.0) * disc[:, :_TOP_K],
        axis=1,
        keepdims=True,
    )
    gain = jnp.exp2(xc) - 1.0

    lam = jnp.zeros((_R, _L), jnp.float32)
    for k in range(_L):
        sx = jnp.sign(xs - xs[:, k : k + 1])
        f1 = -1.0 / (1.0 + jnp.exp(ys - ys[:, k : k + 1]))
        nd = jnp.abs((gain - gain[:, k : k + 1]) * (disc - float(_DISC[k])))
        lam = lam + sx * f1 * nd
    lam_ref[...] = lam / idcg

    grow40 = jax.lax.broadcasted_iota(jnp.int32, (_R, _L), 0) + i * _R
    fidx_ref[...] = yi + grow40 * _N


def _tc_extract(x_similarity, y_similarity):
    grid = (_N // _R,)
    return pl.pallas_call(
        _extract_kernel,
        grid=grid,
        in_specs=[
            pl.BlockSpec((_R, _N), lambda i: (i, 0)),
            pl.BlockSpec((_R, _N), lambda i: (i, 0)),
        ],
        out_specs=[
            pl.BlockSpec((_R, _L), lambda i: (i, 0)),
            pl.BlockSpec((_R, _L), lambda i: (i, 0)),
        ],
        out_shape=[
            jax.ShapeDtypeStruct((_N, _L), jnp.float32),
            jax.ShapeDtypeStruct((_N, _L), jnp.int32),
        ],
        compiler_params=pltpu.CompilerParams(
            dimension_semantics=("parallel",)
        ),
    )(x_similarity, y_similarity)


_NW = 32
_ROWS_PER_W = _N // _NW
_VALS_PER_W = _ROWS_PER_W * _L
_SCAT_CHUNK = 128
_ZCHUNK = 16384
_SLAB = _ROWS_PER_W * _N


def _sc_scatter(fidx_flat, lam_flat, zchunk):
    mesh = plsc.VectorSubcoreMesh(core_axis_name="c", subcore_axis_name="s")

    @functools.partial(
        pl.kernel,
        out_type=jax.ShapeDtypeStruct((_N * _N,), jnp.float32),
        mesh=mesh,
        scratch_types=[
            pltpu.VMEM((_VALS_PER_W,), jnp.int32),
            pltpu.VMEM((_VALS_PER_W,), jnp.float32),
            pltpu.VMEM((_ZCHUNK,), jnp.float32),
            pltpu.SemaphoreType.DMA,
            pltpu.SemaphoreType.DMA,
        ],
    )
    def scatter_kernel(
        fidx_hbm, lam_hbm, z_hbm, out_hbm, idx_v, val_v, z_v, zsem, ssem
    ):
        wid = lax.axis_index("s") * 2 + lax.axis_index("c")
        pltpu.sync_copy(z_hbm, z_v)
        slab = wid * _SLAB
        zcopies = [
            pltpu.async_copy(
                z_v, out_hbm.at[pl.ds(slab + k * _ZCHUNK, _ZCHUNK)], zsem
            )
            for k in range(_SLAB // _ZCHUNK)
        ]
        base = wid * _VALS_PER_W
        pltpu.sync_copy(fidx_hbm.at[pl.ds(base, _VALS_PER_W)], idx_v)
        pltpu.sync_copy(lam_hbm.at[pl.ds(base, _VALS_PER_W)], val_v)
        for c in zcopies:
            c.wait()
        scopies = [
            pltpu.async_copy(
                val_v.at[pl.ds(c * _SCAT_CHUNK, _SCAT_CHUNK)],
                out_hbm.at[idx_v.at[pl.ds(c * _SCAT_CHUNK, _SCAT_CHUNK)]],
                ssem,
            )
            for c in range(_VALS_PER_W // _SCAT_CHUNK)
        ]
        for c in scopies:
            c.wait()

    return scatter_kernel(fidx_flat, lam_flat, zchunk)


@jax.jit
def kernel(x_similarity, y_similarity):
    lam, fidx = _tc_extract(x_similarity, y_similarity)
    zchunk = jnp.zeros((_ZCHUNK,), jnp.float32)
    out_flat = _sc_scatter(fidx.reshape(-1), lam.reshape(-1), zchunk)
    return out_flat.reshape(_N, _N)
